# baseline (speedup 1.0000x reference)
import functools

import jax
import jax.numpy as jnp
from jax import lax
from jax.experimental import pallas as pl
from jax.experimental.pallas import tpu as pltpu
from jax.experimental.pallas import tpu_sc as plsc

NUM_CORES = 2
NUM_SUBCORES = 16
NUM_WORKERS = NUM_CORES * NUM_SUBCORES
BATCH = 16384
DIM = 128
NJ = DIM // 16
BW = BATCH // NUM_WORKERS
CHUNK = 64
NCHUNK = BW // CHUNK


def _body(idxht_hbm, idxr_hbm, e_hbm, rw_hbm, out_hbm,
          idxht, idxr, eb, rwb, outb, sems):
    wid = lax.axis_index("s") * NUM_CORES + lax.axis_index("c")

    pltpu.sync_copy(idxht_hbm.at[wid], idxht)
    pltpu.sync_copy(idxr_hbm.at[wid], idxr)

    lanes = lax.iota(jnp.int32, 16)

    def issue(k):
        par = k % 2
        sem = sems.at[par]
        pltpu.async_copy(e_hbm.at[idxht.at[k]], eb.at[par], sem)
        pltpu.async_copy(rw_hbm.at[idxr.at[k]], rwb.at[par], sem)

    def drain(k):
        par = k % 2
        pltpu.make_async_copy(
            e_hbm.at[pl.ds(0, 2 * CHUNK)], eb.at[par], sems.at[par]).wait()
        pltpu.make_async_copy(
            rw_hbm.at[pl.ds(0, CHUNK)], rwb.at[par], sems.at[par]).wait()

    def compute(k):
        par = k % 2

        def group_body(g, carry2):
            outv = jnp.zeros((16,), jnp.float32)
            for rr in range(16):
                i = g * 16 + rr
                d = []
                w = []
                s1v = jnp.zeros((16,), jnp.float32)
                s2v = jnp.zeros((16,), jnp.float32)
                for j in range(NJ):
                    sl = pl.ds(j * 16, 16)
                    dv = eb[par, i, sl] - eb[par, CHUNK + i, sl]
                    wv = rwb[par, i, pl.ds(DIM + j * 16, 16)]
                    d.append(dv)
                    w.append(wv)
                    s1v = s1v + dv * wv
                    s2v = s2v + wv * wv
                s1 = jnp.broadcast_to(jnp.sum(s1v), (16,))
                s2 = jnp.broadcast_to(jnp.sum(s2v), (16,))
                coeff = s1 / jnp.maximum(s2, 1e-24)
                accv = jnp.zeros((16,), jnp.float32)
                for j in range(NJ):
                    rv = rwb[par, i, pl.ds(j * 16, 16)]
                    accv = accv + jnp.abs(d[j] + rv - coeff * w[j])
                acc = jnp.broadcast_to(jnp.sum(accv), (16,))
                outv = jnp.where(lanes == rr, acc, outv)
            base = pl.multiple_of(k * CHUNK + g * 16, 16)
            outb[pl.ds(base, 16)] = outv
            return carry2

        lax.fori_loop(0, CHUNK // 16, group_body, 0)

    issue(0)

    def chunk_body(k, carry):
        @pl.when(k + 1 < NCHUNK)
        def _():
            issue(k + 1)

        drain(k)
        compute(k)
        return carry

    lax.fori_loop(0, NCHUNK, chunk_body, 0)
    pltpu.sync_copy(outb, out_hbm.at[pl.ds(pl.multiple_of(wid * BW, 8), BW)])


@jax.jit
def kernel(h, r, t, E, R, W):
    mesh = plsc.VectorSubcoreMesh(core_axis_name="c", subcore_axis_name="s")
    ebuf = pltpu.VMEM((2, 2 * CHUNK, DIM), jnp.float32)
    rwbuf = pltpu.VMEM((2, CHUNK, 2 * DIM), jnp.float32)
    kfn = pl.kernel(
        _body,
        out_type=jax.ShapeDtypeStruct((BATCH,), jnp.float32),
        mesh=mesh,
        compiler_params=pltpu.CompilerParams(needs_layout_passes=False),
        scratch_types=[
            pltpu.VMEM((NCHUNK, 2 * CHUNK), jnp.int32),
            pltpu.VMEM((NCHUNK, CHUNK), jnp.int32),
            ebuf, rwbuf,
            pltpu.VMEM((BW,), jnp.float32),
            pltpu.SemaphoreType.DMA((2,)),
        ],
    )
    h3 = h.reshape(NUM_WORKERS, NCHUNK, CHUNK)
    t3 = t.reshape(NUM_WORKERS, NCHUNK, CHUNK)
    idxht = jnp.concatenate([h3, t3], axis=2)
    idxr = r.reshape(NUM_WORKERS, NCHUNK, CHUNK)
    rw = jnp.concatenate([R, W], axis=1)
    return kfn(idxht, idxr, E, rw)


# speedup vs baseline: 1.0809x; 1.0809x over previous
import functools

import jax
import jax.numpy as jnp
from jax import lax
from jax.experimental import pallas as pl
from jax.experimental.pallas import tpu as pltpu
from jax.experimental.pallas import tpu_sc as plsc

NUM_CORES = 2
NUM_SUBCORES = 16
NUM_WORKERS = NUM_CORES * NUM_SUBCORES
BATCH = 16384
DIM = 128
NJ = DIM // 16
BW = BATCH // NUM_WORKERS
CHUNK = 64
NCHUNK = BW // CHUNK


def _body(idx_hbm, e_hbm, rel_hbm, w_hbm, out_hbm,
          idxv, hb, tb, rb, wb, outb, sems):
    wid = lax.axis_index("s") * NUM_CORES + lax.axis_index("c")

    pltpu.sync_copy(idx_hbm.at[wid], idxv)

    lanes = lax.iota(jnp.int32, 16)

    def issue(k):
        par = k % 2
        sem = sems.at[par]
        pltpu.async_copy(e_hbm.at[idxv.at[0, k]], hb.at[par], sem)
        pltpu.async_copy(e_hbm.at[idxv.at[1, k]], tb.at[par], sem)
        pltpu.async_copy(rel_hbm.at[idxv.at[2, k]], rb.at[par], sem)
        pltpu.async_copy(w_hbm.at[idxv.at[2, k]], wb.at[par], sem)

    def drain(k):
        par = k % 2
        for b in (hb, tb, rb, wb):
            pltpu.make_async_copy(
                e_hbm.at[pl.ds(0, CHUNK)], b.at[par], sems.at[par]).wait()

    def compute(k):
        par = k % 2

        def group_body(g, carry2):
            outv = jnp.zeros((16,), jnp.float32)
            for rr in range(16):
                i = g * 16 + rr
                d = []
                w = []
                s1v = jnp.zeros((16,), jnp.float32)
                s2v = jnp.zeros((16,), jnp.float32)
                for j in range(NJ):
                    sl = pl.ds(j * 16, 16)
                    dv = hb[par, i, sl] - tb[par, i, sl]
                    wv = wb[par, i, sl]
                    d.append(dv)
                    w.append(wv)
                    s1v = s1v + dv * wv
                    s2v = s2v + wv * wv
                s1 = jnp.broadcast_to(jnp.sum(s1v), (16,))
                s2 = jnp.broadcast_to(jnp.sum(s2v), (16,))
                coeff = s1 / jnp.maximum(s2, 1e-24)
                accv = jnp.zeros((16,), jnp.float32)
                for j in range(NJ):
                    rv = rb[par, i, pl.ds(j * 16, 16)]
                    accv = accv + jnp.abs(d[j] + rv - coeff * w[j])
                acc = jnp.broadcast_to(jnp.sum(accv), (16,))
                outv = jnp.where(lanes == rr, acc, outv)
            base = pl.multiple_of(k * CHUNK + g * 16, 16)
            outb[pl.ds(base, 16)] = outv
            return carry2

        lax.fori_loop(0, CHUNK // 16, group_body, 0)

    issue(0)

    def chunk_body(k, carry):
        @pl.when(k + 1 < NCHUNK)
        def _():
            issue(k + 1)

        drain(k)
        compute(k)
        return carry

    lax.fori_loop(0, NCHUNK, chunk_body, 0)
    pltpu.sync_copy(outb, out_hbm.at[pl.ds(pl.multiple_of(wid * BW, 8), BW)])


@jax.jit
def kernel(h, r, t, E, R, W):
    mesh = plsc.VectorSubcoreMesh(core_axis_name="c", subcore_axis_name="s")
    buf = pltpu.VMEM((2, CHUNK, DIM), jnp.float32)
    kfn = pl.kernel(
        _body,
        out_type=jax.ShapeDtypeStruct((BATCH,), jnp.float32),
        mesh=mesh,
        compiler_params=pltpu.CompilerParams(needs_layout_passes=False),
        scratch_types=[
            pltpu.VMEM((3, NCHUNK, CHUNK), jnp.int32),
            buf, buf, buf, buf,
            pltpu.VMEM((BW,), jnp.float32),
            pltpu.SemaphoreType.DMA((2,)),
        ],
    )
    idx = jnp.stack([h, t, r]).reshape(3, NUM_WORKERS, NCHUNK, CHUNK)
    idx = idx.transpose(1, 0, 2, 3)
    return kfn(idx, E, R, W)
